# baseline (speedup 1.0000x reference)
import functools

import jax
import jax.numpy as jnp
from jax import lax
from jax.experimental import pallas as pl
from jax.experimental.pallas import tpu as pltpu
from jax.experimental.pallas import tpu_sc as plsc

NC = 2
NS = 16
NW = NC * NS

CR = 8
NB = 2


def _gather_kernel(rows_w, n_chunks, S, D, data_hbm, table_hbm, out_hbm,
                   idx_v, rows_a, rows_b, gsem_a, gsem_b, ssem_a, ssem_b):
    bufs = (rows_a, rows_b)
    gs = (gsem_a, gsem_b)
    ss = (ssem_a, ssem_b)
    wid = lax.axis_index("s") * NC + lax.axis_index("c")
    base = wid * rows_w

    pltpu.sync_copy(data_hbm.at[pl.ds(base, rows_w)], idx_v)

    def outer(o, carry):
        g0 = o * NB
        for b in range(NB):
            off = (g0 + b) * CR
            for r in range(CR):
                pltpu.async_copy(table_hbm.at[idx_v.at[off + r]],
                                 bufs[b].at[r], gs[b])
        for b in range(NB):
            off = (g0 + b) * CR
            for r in range(CR):
                pltpu.make_async_copy(table_hbm.at[idx_v.at[off + r]],
                                      bufs[b].at[r], gs[b]).wait()
            for r in range(CR):
                pltpu.async_copy(bufs[b].at[r],
                                 out_hbm.at[base + off + r], ss[b])
        for b in range(NB):
            off = (g0 + b) * CR
            for r in range(CR):
                pltpu.make_async_copy(bufs[b].at[r],
                                      out_hbm.at[base + off + r],
                                      ss[b]).wait()
        return carry

    lax.fori_loop(0, n_chunks // NB, outer, 0)


def kernel(data, table):
    B0, S = data.shape
    V, D = table.shape

    rows_w = B0 // NW
    n_chunks = rows_w // CR
    assert rows_w * NW == B0 and n_chunks * CR == rows_w
    assert n_chunks % NB == 0

    mesh = plsc.VectorSubcoreMesh(core_axis_name="c", subcore_axis_name="s")
    run = functools.partial(
        pl.kernel,
        out_type=jax.ShapeDtypeStruct((B0, S, D), jnp.float32),
        mesh=mesh,
        scratch_types=[
            pltpu.VMEM((rows_w, S), jnp.int32),
            pltpu.VMEM((CR, S, D), jnp.float32),
            pltpu.VMEM((CR, S, D), jnp.float32),
            pltpu.SemaphoreType.DMA,
            pltpu.SemaphoreType.DMA,
            pltpu.SemaphoreType.DMA,
            pltpu.SemaphoreType.DMA,
        ],
        compiler_params=pltpu.CompilerParams(use_tc_tiling_on_sc=False),
    )(functools.partial(_gather_kernel, rows_w, n_chunks, S, D))
    return run(data, table)


# speedup vs baseline: 1.4609x; 1.4609x over previous
import functools

import jax
import jax.numpy as jnp
from jax import lax
from jax.experimental import pallas as pl
from jax.experimental.pallas import tpu as pltpu
from jax.experimental.pallas import tpu_sc as plsc

NC = 2
NS = 16
NW = NC * NS

CR = 8
NB = 2


def _gather_kernel(rows_w, n_chunks, S, D, data_hbm, table_hbm, out_hbm,
                   idx_v, rows_a, rows_b, gsem_a, gsem_b, ssem_a, ssem_b):
    bufs = (rows_a, rows_b)
    gs = (gsem_a, gsem_b)
    ss = (ssem_a, ssem_b)
    wid = lax.axis_index("s") * NC + lax.axis_index("c")
    base = wid * rows_w

    pltpu.sync_copy(data_hbm.at[pl.ds(base, rows_w)], idx_v)

    def outer(o, carry):
        g0 = o * NB
        for b in range(NB):
            off = (g0 + b) * CR
            for r in range(CR):
                pltpu.async_copy(table_hbm.at[idx_v.at[off + r]],
                                 bufs[b].at[r], gs[b])
        for b in range(NB):
            off = (g0 + b) * CR
            for r in range(CR):
                pltpu.make_async_copy(table_hbm.at[idx_v.at[off + r]],
                                      bufs[b].at[r], gs[b]).wait()
            for r in range(CR):
                pltpu.async_copy(bufs[b].at[r],
                                 out_hbm.at[base + off + r], ss[b])
        for b in range(NB):
            off = (g0 + b) * CR
            for r in range(CR):
                pltpu.make_async_copy(bufs[b].at[r],
                                      out_hbm.at[base + off + r],
                                      ss[b]).wait()
        return carry

    lax.fori_loop(0, n_chunks // NB, outer, 0)


VB = 16384


def _fmt_kernel(vb, tT_ref, o_ref, xt_ref):
    xt_ref[...] = tT_ref[...].T
    o_ref[:, 0:64] = xt_ref[0::2, :]
    o_ref[:, 64:128] = xt_ref[1::2, :]


def _format_table(table):
    V, D = table.shape
    nvb = (V + VB - 1) // VB
    t2 = pl.pallas_call(
        functools.partial(_fmt_kernel, VB),
        grid=(nvb,),
        in_specs=[pl.BlockSpec((D, VB), lambda i: (0, i))],
        out_specs=pl.BlockSpec((VB // 2, 128), lambda i: (i, 0)),
        out_shape=jax.ShapeDtypeStruct((V // 2, 128), jnp.float32),
        scratch_shapes=[pltpu.VMEM((VB, 64), jnp.float32)],
    )(table.T)
    return jnp.reshape(t2, (V, D))


def kernel(data, table):
    B0, S = data.shape
    V, D = table.shape

    rows_w = B0 // NW
    n_chunks = rows_w // CR
    assert rows_w * NW == B0 and n_chunks * CR == rows_w
    assert n_chunks % NB == 0

    table_lin = _format_table(table)

    mesh = plsc.VectorSubcoreMesh(core_axis_name="c", subcore_axis_name="s")
    run = functools.partial(
        pl.kernel,
        out_type=jax.ShapeDtypeStruct((B0, S, D), jnp.float32),
        mesh=mesh,
        scratch_types=[
            pltpu.VMEM((rows_w, S), jnp.int32),
            pltpu.VMEM((CR, S, D), jnp.float32),
            pltpu.VMEM((CR, S, D), jnp.float32),
            pltpu.SemaphoreType.DMA,
            pltpu.SemaphoreType.DMA,
            pltpu.SemaphoreType.DMA,
            pltpu.SemaphoreType.DMA,
        ],
        compiler_params=pltpu.CompilerParams(use_tc_tiling_on_sc=False),
    )(functools.partial(_gather_kernel, rows_w, n_chunks, S, D))
    return run(data, table_lin)


# speedup vs baseline: 1.4939x; 1.0226x over previous
import functools

import jax
import jax.numpy as jnp
from jax import lax
from jax.experimental import pallas as pl
from jax.experimental.pallas import tpu as pltpu
from jax.experimental.pallas import tpu_sc as plsc

NC = 2
NS = 16
NW = NC * NS

CR = 8
NB = 2


def _gather_kernel(rows_w, n_chunks, S, D, data_hbm, table_hbm, out_hbm,
                   idx_v, rows_a, rows_b, gsem_a, gsem_b, ssem_a, ssem_b):
    bufs = (rows_a, rows_b)
    gs = (gsem_a, gsem_b)
    ss = (ssem_a, ssem_b)
    wid = lax.axis_index("s") * NC + lax.axis_index("c")
    base = wid * rows_w

    pltpu.sync_copy(data_hbm.at[pl.ds(base, rows_w)], idx_v)

    def outer(o, carry):
        g0 = o * NB
        for b in range(NB):
            off = (g0 + b) * CR
            for r in range(CR):
                pltpu.async_copy(table_hbm.at[idx_v.at[off + r]],
                                 bufs[b].at[r], gs[b])
        for b in range(NB):
            off = (g0 + b) * CR
            for r in range(CR):
                pltpu.make_async_copy(table_hbm.at[idx_v.at[off + r]],
                                      bufs[b].at[r], gs[b]).wait()
            for r in range(CR):
                pltpu.async_copy(bufs[b].at[r],
                                 out_hbm.at[base + off + r], ss[b])
        for b in range(NB):
            off = (g0 + b) * CR
            for r in range(CR):
                pltpu.make_async_copy(bufs[b].at[r],
                                      out_hbm.at[base + off + r],
                                      ss[b]).wait()
        return carry

    lax.fori_loop(0, n_chunks // NB, outer, 0)


VB = 16384


def _fmt_kernel(vb, tT_ref, o_ref, xt_ref):
    x = tT_ref[...]
    eye = (lax.broadcasted_iota(jnp.int32, (64, 64), 0)
           == lax.broadcasted_iota(jnp.int32, (64, 64), 1)).astype(jnp.float32)
    xt_ref[...] = lax.dot_general(x, eye, (((0,), (0,)), ((), ())),
                                  preferred_element_type=jnp.float32)
    o_ref[:, 0:64] = xt_ref[0::2, :]
    o_ref[:, 64:128] = xt_ref[1::2, :]


def _format_table(table):
    V, D = table.shape
    nvb = (V + VB - 1) // VB
    t2 = pl.pallas_call(
        functools.partial(_fmt_kernel, VB),
        grid=(nvb,),
        in_specs=[pl.BlockSpec((D, VB), lambda i: (0, i))],
        out_specs=pl.BlockSpec((VB // 2, 128), lambda i: (i, 0)),
        out_shape=jax.ShapeDtypeStruct((V // 2, 128), jnp.float32),
        scratch_shapes=[pltpu.VMEM((VB, 64), jnp.float32)],
    )(table.T)
    return jnp.reshape(t2, (V, D))


def kernel(data, table):
    B0, S = data.shape
    V, D = table.shape

    rows_w = B0 // NW
    n_chunks = rows_w // CR
    assert rows_w * NW == B0 and n_chunks * CR == rows_w
    assert n_chunks % NB == 0

    table_lin = _format_table(table)

    mesh = plsc.VectorSubcoreMesh(core_axis_name="c", subcore_axis_name="s")
    run = functools.partial(
        pl.kernel,
        out_type=jax.ShapeDtypeStruct((B0, S, D), jnp.float32),
        mesh=mesh,
        scratch_types=[
            pltpu.VMEM((rows_w, S), jnp.int32),
            pltpu.VMEM((CR, S, D), jnp.float32),
            pltpu.VMEM((CR, S, D), jnp.float32),
            pltpu.SemaphoreType.DMA,
            pltpu.SemaphoreType.DMA,
            pltpu.SemaphoreType.DMA,
            pltpu.SemaphoreType.DMA,
        ],
        compiler_params=pltpu.CompilerParams(use_tc_tiling_on_sc=False),
    )(functools.partial(_gather_kernel, rows_w, n_chunks, S, D))
    return run(data, table_lin)
